# baseline (speedup 1.0000x reference)
import functools

import jax
import jax.numpy as jnp
from jax import lax
from jax.experimental import pallas as pl
from jax.experimental.pallas import tpu as pltpu
from jax.experimental.pallas import tpu_sc as plsc

H = 1024
NSP = 8
CORPUS = 512
ROWS = CORPUS + NSP
BATCH = 32
SC_BATCHES = 18
W2_ROWS = NSP * H


def _mlp_body(w1_ref, b1_ref, w2_ref, b2_ref, sp_ref, h_ref):
    r = pl.program_id(0)

    @pl.when(r == 0)
    def _():
        ones = jnp.ones((8, H), jnp.float32)
        h = lax.dot_general(ones, w1_ref[...], (((1,), (1,)), ((), ())),
                            preferred_element_type=jnp.float32)
        h_ref[...] = jax.nn.relu(h + b1_ref[...][None, :])

    res = lax.dot_general(h_ref[...], w2_ref[...], (((1,), (1,)), ((), ())),
                          preferred_element_type=jnp.float32)
    sp_ref[pl.ds(r, 1), :] = res[0:1, :] + b2_ref[...][None, :]


def _soft_prompt(W1, b1, W2, b2):
    return pl.pallas_call(
        _mlp_body,
        grid=(NSP,),
        in_specs=[
            pl.BlockSpec((H, H), lambda r: (0, 0)),
            pl.BlockSpec((H,), lambda r: (0,)),
            pl.BlockSpec((H, H), lambda r: (r, 0)),
            pl.BlockSpec((H,), lambda r: (r,)),
        ],
        out_specs=pl.BlockSpec((NSP, H), lambda r: (0, 0)),
        out_shape=jax.ShapeDtypeStruct((NSP, H), jnp.float32),
        scratch_shapes=[pltpu.VMEM((8, H), jnp.float32)],
    )(W1, b1, W2, b2)


def _sc_broadcast_de(de):
    mesh = plsc.ScalarSubcoreMesh(axis_name="c", num_cores=2)
    per_core = SC_BATCHES // 2

    @functools.partial(
        pl.kernel,
        out_type=jax.ShapeDtypeStruct((BATCH, ROWS, H), jnp.float32),
        mesh=mesh,
        scratch_types=[
            pltpu.VMEM_SHARED((CORPUS, H), jnp.float32),
            pltpu.SemaphoreType.DMA,
        ],
    )
    def body(de_hbm, out_hbm, shared, sem):
        c = lax.axis_index("c")
        pltpu.sync_copy(de_hbm, shared)
        copies = []
        for i in range(per_core):
            b = c * per_core + i
            copies.append(
                pltpu.async_copy(shared, out_hbm.at[b, pl.ds(0, CORPUS)], sem))
        for cp in copies:
            cp.wait()

    return body(de)


def _finish_body(out_alias_ref, de_ref, sp_ref, out_ref, sem):
    del out_alias_ref
    copies = []
    for b in range(SC_BATCHES, BATCH):
        copies.append(
            pltpu.make_async_copy(de_ref, out_ref.at[b, pl.ds(0, CORPUS)], sem))
    for b in range(BATCH):
        copies.append(
            pltpu.make_async_copy(sp_ref, out_ref.at[b, pl.ds(CORPUS, NSP)], sem))
    for cp in copies:
        cp.start()
    for cp in copies:
        cp.wait()


def _tc_finish(out1, de, sp):
    return pl.pallas_call(
        _finish_body,
        grid=(1,),
        in_specs=[
            pl.BlockSpec(memory_space=pl.ANY),
            pl.BlockSpec((CORPUS, H), lambda i: (0, 0)),
            pl.BlockSpec((NSP, H), lambda i: (0, 0)),
        ],
        out_specs=pl.BlockSpec(memory_space=pl.ANY),
        out_shape=jax.ShapeDtypeStruct((BATCH, ROWS, H), jnp.float32),
        input_output_aliases={0: 0},
        scratch_shapes=[pltpu.SemaphoreType.DMA],
    )(out1, de, sp)


def kernel(input_ids, dataset_embeddings, W1, b1, W2, b2):
    del input_ids
    de = dataset_embeddings.astype(jnp.float32)
    sp = _soft_prompt(W1, b1, W2, b2)
    out = _sc_broadcast_de(de)
    out = _tc_finish(out, de, sp)
    return out


# speedup vs baseline: 1.0202x; 1.0202x over previous
import functools

import jax
import jax.numpy as jnp
from jax import lax
from jax.experimental import pallas as pl
from jax.experimental.pallas import tpu as pltpu
from jax.experimental.pallas import tpu_sc as plsc

H = 1024
NSP = 8
CORPUS = 512
ROWS = CORPUS + NSP
BATCH = 32
SC_BATCHES = 16


def _mlp_body(w1_ref, b1_ref, w2_ref, b2_ref, sp_ref, h_ref):
    r = pl.program_id(0)

    @pl.when(r == 0)
    def _():
        ones = jnp.ones((8, H), jnp.float32)
        h = lax.dot_general(ones, w1_ref[...], (((1,), (1,)), ((), ())),
                            preferred_element_type=jnp.float32)
        h_ref[...] = jax.nn.relu(h + b1_ref[...][None, :])

    res = lax.dot_general(h_ref[...], w2_ref[...], (((1,), (1,)), ((), ())),
                          preferred_element_type=jnp.float32)
    sp_ref[pl.ds(r, 1), :] = res[0:1, :] + b2_ref[...][None, :]


def _soft_prompt(W1, b1, W2, b2):
    return pl.pallas_call(
        _mlp_body,
        grid=(NSP,),
        in_specs=[
            pl.BlockSpec((H, H), lambda r: (0, 0)),
            pl.BlockSpec((H,), lambda r: (0,)),
            pl.BlockSpec((H, H), lambda r: (r, 0)),
            pl.BlockSpec((H,), lambda r: (r,)),
        ],
        out_specs=pl.BlockSpec((NSP, H), lambda r: (0, 0)),
        out_shape=jax.ShapeDtypeStruct((NSP, H), jnp.float32),
        scratch_shapes=[pltpu.VMEM((8, H), jnp.float32)],
    )(W1, b1, W2, b2)


def _sc_broadcast_de(de):
    mesh = plsc.ScalarSubcoreMesh(axis_name="c", num_cores=2)
    per_core = SC_BATCHES // 2

    @functools.partial(
        pl.kernel,
        out_type=jax.ShapeDtypeStruct((BATCH, ROWS, H), jnp.float32),
        mesh=mesh,
        scratch_types=[
            pltpu.VMEM_SHARED((CORPUS, H), jnp.float32),
            pltpu.SemaphoreType.DMA,
        ],
    )
    def body(de_hbm, out_hbm, shared, sem):
        c = lax.axis_index("c")
        pltpu.sync_copy(de_hbm, shared)
        copies = []
        for i in range(per_core):
            b = c * per_core + i
            copies.append(
                pltpu.async_copy(shared, out_hbm.at[b, pl.ds(0, CORPUS)], sem))
        for cp in copies:
            cp.wait()

    return body(de)


def _finish_body(out_alias_ref, de_ref, sp_ref, out_ref, sem):
    del out_alias_ref
    copies = []
    for b in range(SC_BATCHES, BATCH):
        copies.append(
            pltpu.make_async_copy(de_ref, out_ref.at[b, pl.ds(0, CORPUS)], sem))
    for b in range(BATCH):
        copies.append(
            pltpu.make_async_copy(sp_ref, out_ref.at[b, pl.ds(CORPUS, NSP)], sem))
    for cp in copies:
        cp.start()
    for cp in copies:
        cp.wait()


def _tc_finish(out1, de, sp):
    return pl.pallas_call(
        _finish_body,
        grid=(1,),
        in_specs=[
            pl.BlockSpec(memory_space=pl.ANY),
            pl.BlockSpec((CORPUS, H), lambda i: (0, 0)),
            pl.BlockSpec((NSP, H), lambda i: (0, 0)),
        ],
        out_specs=pl.BlockSpec(memory_space=pl.ANY),
        out_shape=jax.ShapeDtypeStruct((BATCH, ROWS, H), jnp.float32),
        input_output_aliases={0: 0},
        scratch_shapes=[pltpu.SemaphoreType.DMA],
    )(out1, de, sp)


def kernel(input_ids, dataset_embeddings, W1, b1, W2, b2):
    del input_ids
    de = dataset_embeddings.astype(jnp.float32)
    sp = _soft_prompt(W1, b1, W2, b2)
    out = _sc_broadcast_de(de)
    out = _tc_finish(out, de, sp)
    return out
